# baseline (speedup 1.0000x reference)
import functools

import jax
import jax.numpy as jnp
from jax import lax
from jax.experimental import pallas as pl
from jax.experimental.pallas import tpu as pltpu
from jax.experimental.pallas import tpu_sc as plsc

_B, _S, _D = 4, 4096, 768
_N = _B * _S
_NC, _NS = 2, 16
_NW = _NC * _NS
_PS = _S // _NW
_C = 16
_NG = _PS // _C
_NV = _D // 16
_NSLOT = 8

_mesh = plsc.VectorSubcoreMesh(core_axis_name="c", subcore_axis_name="s")


@functools.partial(
    pl.kernel,
    mesh=_mesh,
    out_type=jax.ShapeDtypeStruct((_N, _D), jnp.float32),
    scratch_types=[
        pltpu.VMEM((_B, _PS), jnp.int32),
        pltpu.VMEM((_C, _D), jnp.float32),
        pltpu.VMEM((_NSLOT, _C, _D), jnp.float32),
        pltpu.SemaphoreType.DMA,
        pltpu.SemaphoreType.DMA,
        pltpu.SemaphoreType.DMA,
        pltpu.SemaphoreType.DMA,
    ],
)
def _emb_lookup(tok_hbm, emb_hbm, pos_hbm, out_hbm,
                idx_v, pos_v, ring, sem_i, sem_p, sem_g, sem_s):
    wid = lax.axis_index("s") * _NC + lax.axis_index("c")
    s_w = wid * _PS

    def slot(g, b):
        return 4 * (g % 2) + b

    def gathers(g):
        return [
            pltpu.async_copy(
                emb_hbm.at[idx_v.at[b, pl.ds(g * _C, _C)]],
                ring.at[slot(g, b)], sem_g)
            for b in range(_B)
        ]

    def pos_load(g):
        return pltpu.async_copy(
            pos_hbm.at[pl.ds(s_w + g * _C, _C)], pos_v, sem_p)

    idx_descs = [
        pltpu.async_copy(tok_hbm.at[pl.ds(b * _S + s_w, _PS)], idx_v.at[b], sem_i)
        for b in range(_B)
    ]
    p_desc = pos_load(0)
    for d in idx_descs:
        d.wait()
    g_descs = gathers(0)
    s_descs = None

    for g in range(_NG):
        for d in g_descs:
            d.wait()
        if g + 1 < _NG:
            if s_descs is not None:
                for d in s_descs:
                    d.wait()
            next_g_descs = gathers(g + 1)
        p_desc.wait()

        sb = 4 * (g % 2)

        def row_add(i, _):
            for j in range(_NV):
                sl = pl.ds(j * 16, 16)
                v = pos_v[i, sl]
                for b in range(_B):
                    plsc.addupdate(ring.at[sb + b, i, sl], v)
            return 0

        lax.fori_loop(0, _C, row_add, 0)

        s_descs = [
            pltpu.async_copy(
                ring.at[slot(g, b)],
                out_hbm.at[pl.ds(b * _S + s_w + g * _C, _C)], sem_s)
            for b in range(_B)
        ]
        if g + 1 < _NG:
            p_desc = pos_load(g + 1)
            g_descs = next_g_descs
    for d in s_descs:
        d.wait()


def kernel(token_ids, embedding_weight, positional_weight):
    tok = jnp.reshape(token_ids.astype(jnp.int32), (_N,))
    out = _emb_lookup(tok, embedding_weight, positional_weight)
    return jnp.reshape(out, (_B, _S, _D))


# speedup vs baseline: 1.0934x; 1.0934x over previous
import functools

import jax
import jax.numpy as jnp
from jax import lax
from jax.experimental import pallas as pl
from jax.experimental.pallas import tpu as pltpu
from jax.experimental.pallas import tpu_sc as plsc

_B, _S, _D = 4, 4096, 768
_N = _B * _S
_NC, _NS = 2, 16
_NW = _NC * _NS
_PS = _S // _NW
_C = 16
_NG = _PS // _C
_GROWS = _B * _C
_NGG = _S // _C
_NV = _D // 16

_mesh = plsc.VectorSubcoreMesh(core_axis_name="c", subcore_axis_name="s")


@functools.partial(
    pl.kernel,
    mesh=_mesh,
    out_type=jax.ShapeDtypeStruct((_N, _D), jnp.float32),
    scratch_types=[
        pltpu.VMEM((_NG, _GROWS), jnp.int32),
        pltpu.VMEM((2, _C, _D), jnp.float32),
        pltpu.VMEM((2, _GROWS, _D), jnp.float32),
        pltpu.SemaphoreType.DMA,
        pltpu.SemaphoreType.DMA,
        pltpu.SemaphoreType.DMA,
        pltpu.SemaphoreType.DMA,
    ],
)
def _emb_lookup(tokg_hbm, emb_hbm, pos_hbm, out_hbm,
                idx_v, pos_v, ring, sem_i, sem_p, sem_g, sem_s):
    wid = lax.axis_index("s") * _NC + lax.axis_index("c")
    s_w = wid * _PS

    def gather(g):
        return pltpu.async_copy(
            emb_hbm.at[idx_v.at[g]], ring.at[g % 2], sem_g)

    def pos_load(g):
        return pltpu.async_copy(
            pos_hbm.at[pl.ds(s_w + g * _C, _C)], pos_v.at[g % 2], sem_p)

    idx_desc = pltpu.async_copy(
        tokg_hbm.at[pl.ds(wid * _NG, _NG)], idx_v, sem_i)
    p_desc = pos_load(0)
    idx_desc.wait()
    g_desc = gather(0)
    s_descs = None

    for g in range(_NG):
        k = g % 2
        g_desc.wait()
        if g + 1 < _NG:
            if s_descs is not None:
                for d in s_descs:
                    d.wait()
            next_g_desc = gather(g + 1)
        p_desc.wait()
        if g + 1 < _NG:
            next_p_desc = pos_load(g + 1)

        def row_add(i, _):
            for j in range(_NV):
                sl = pl.ds(j * 16, 16)
                v = pos_v[k, i, sl]
                for b in range(_B):
                    plsc.addupdate(ring.at[k, b * _C + i, sl], v)
            return 0

        lax.fori_loop(0, _C, row_add, 0)

        s_descs = [
            pltpu.async_copy(
                ring.at[k, pl.ds(b * _C, _C)],
                out_hbm.at[pl.ds(b * _S + s_w + g * _C, _C)], sem_s)
            for b in range(_B)
        ]
        if g + 1 < _NG:
            g_desc = next_g_desc
            p_desc = next_p_desc
    for d in s_descs:
        d.wait()


def kernel(token_ids, embedding_weight, positional_weight):
    tok_g = (token_ids.astype(jnp.int32)
             .reshape(_B, _NGG, _C).transpose(1, 0, 2).reshape(_NGG, _GROWS))
    out = _emb_lookup(tok_g, embedding_weight, positional_weight)
    return jnp.reshape(out, (_B, _S, _D))
